# baseline (speedup 1.0000x reference)
import functools
import jax
import jax.numpy as jnp
from jax import lax
from jax.experimental import pallas as pl
from jax.experimental.pallas import tpu as pltpu
from jax.experimental.pallas import tpu_sc as plsc

_NC = 2
_NS = 16
_L = 16
_CH = 32
_NBUF = 2


def _sc_add(x, emb):
    B, S, D = x.shape
    nw = _NC * _NS
    rows_per_w = S // nw
    nchunk = rows_per_w // _CH
    nsteps = nchunk * B
    groups = _CH * D // _L
    mesh = plsc.VectorSubcoreMesh(core_axis_name="c", subcore_axis_name="s")

    @functools.partial(
        pl.kernel,
        mesh=mesh,
        out_type=jax.ShapeDtypeStruct((B, S, D), jnp.float32),
        scratch_types=[
            pltpu.VMEM((_NBUF, _CH, D), jnp.float32),
            pltpu.VMEM((_CH, D), jnp.float32),
        ]
        + [pltpu.SemaphoreType.DMA] * _NBUF
        + [pltpu.SemaphoreType.DMA]
        + [pltpu.SemaphoreType.DMA] * _NBUF,
    )
    def body(x_hbm, emb_hbm, out_hbm, x_v, emb_v, *sems):
        lx = sems[:_NBUF]
        le = sems[_NBUF]
        st = sems[_NBUF + 1:]
        wid = lax.axis_index("s") * _NC + lax.axis_index("c")
        base = wid * rows_per_w

        def x_load(t):
            c, b = divmod(t, B)
            buf = t % _NBUF
            return pltpu.async_copy(
                x_hbm.at[b, pl.ds(base + c * _CH, _CH)],
                x_v.at[buf], lx[buf])

        def emb_load(c):
            return pltpu.async_copy(
                emb_hbm.at[pl.ds(base + c * _CH, _CH)], emb_v, le)

        def x_store(t):
            c, b = divmod(t, B)
            buf = t % _NBUF
            return pltpu.async_copy(
                x_v.at[buf],
                out_hbm.at[b, pl.ds(base + c * _CH, _CH)], st[buf])

        h_e = emb_load(0)
        h_x = {t: x_load(t) for t in range(min(_NBUF - 1, nsteps))}
        h_e.wait()
        h_st = {}

        for t in range(nsteps):
            c, b = divmod(t, B)
            buf = t % _NBUF
            h_x.pop(t).wait()
            if b == 0 and c > 0:
                emb_load(c).wait()

            @plsc.parallel_loop(0, groups, unroll=16)
            def _(i):
                r = i // (D // _L)
                j = (i % (D // _L)) * _L
                plsc.addupdate(
                    x_v.at[buf, r, pl.ds(j, _L)],
                    emb_v[r, pl.ds(j, _L)])

            h_st[t] = x_store(t)
            u = t + _NBUF - 1
            if u < nsteps:
                if u - _NBUF >= 0:
                    h_st.pop(u - _NBUF).wait()
                h_x[u] = x_load(u)

        for t in sorted(h_st):
            h_st.pop(t).wait()

    return body(x, emb)


def kernel(x, emb):
    return _sc_add(x, emb)


# speedup vs baseline: 1.4578x; 1.4578x over previous
import functools
import jax
import jax.numpy as jnp
from jax import lax
from jax.experimental import pallas as pl
from jax.experimental.pallas import tpu as pltpu
from jax.experimental.pallas import tpu_sc as plsc

_NC = 2
_NS = 16
_L = 16
_CH = 16
_NBUF = 5


def _sc_add(x, emb):
    B, S, D = x.shape
    nw = _NC * _NS
    rows_per_w = S // nw
    nchunk = rows_per_w // _CH
    nsteps = nchunk * B
    groups = _CH * D // _L
    mesh = plsc.VectorSubcoreMesh(core_axis_name="c", subcore_axis_name="s")

    @functools.partial(
        pl.kernel,
        mesh=mesh,
        out_type=jax.ShapeDtypeStruct((B, S, D), jnp.float32),
        scratch_types=[
            pltpu.VMEM((_NBUF, _CH, D), jnp.float32),
            pltpu.VMEM((2, _CH, D), jnp.float32),
        ]
        + [pltpu.SemaphoreType.DMA] * _NBUF
        + [pltpu.SemaphoreType.DMA] * 2
        + [pltpu.SemaphoreType.DMA] * _NBUF,
    )
    def body(x_hbm, emb_hbm, out_hbm, x_v, emb_v, *sems):
        lx = sems[:_NBUF]
        le = sems[_NBUF:_NBUF + 2]
        st = sems[_NBUF + 2:]
        wid = lax.axis_index("s") * _NC + lax.axis_index("c")
        base = wid * rows_per_w

        def x_load(t):
            c, b = divmod(t, B)
            buf = t % _NBUF
            return pltpu.async_copy(
                x_hbm.at[b, pl.ds(base + c * _CH, _CH)],
                x_v.at[buf], lx[buf])

        def emb_load(c):
            buf = c % 2
            return pltpu.async_copy(
                emb_hbm.at[pl.ds(base + c * _CH, _CH)],
                emb_v.at[buf], le[buf])

        def x_store(t):
            c, b = divmod(t, B)
            buf = t % _NBUF
            return pltpu.async_copy(
                x_v.at[buf],
                out_hbm.at[b, pl.ds(base + c * _CH, _CH)], st[buf])

        h_e = emb_load(0)
        h_x = {t: x_load(t) for t in range(min(_NBUF - 1, nsteps))}
        h_e.wait()
        h_st = {}

        for t in range(nsteps):
            c, b = divmod(t, B)
            buf = t % _NBUF
            h_x.pop(t).wait()
            if b == 0 and c + 1 < nchunk:
                h_e = emb_load(c + 1)
            if b == B - 1 and c + 1 < nchunk:
                h_e.wait()

            ebuf = c % 2

            @plsc.parallel_loop(0, groups, unroll=16)
            def _(i):
                r = i // (D // _L)
                j = (i % (D // _L)) * _L
                plsc.addupdate(
                    x_v.at[buf, r, pl.ds(j, _L)],
                    emb_v[ebuf, r, pl.ds(j, _L)])

            h_st[t] = x_store(t)
            u = t + _NBUF - 1
            if u < nsteps:
                if u - _NBUF >= 0:
                    h_st.pop(u - _NBUF).wait()
                h_x[u] = x_load(u)

        for t in sorted(h_st):
            h_st.pop(t).wait()

    return body(x, emb)


def kernel(x, emb):
    return _sc_add(x, emb)
